# baseline (speedup 1.0000x reference)
import jax
import jax.numpy as jnp
from jax import lax
from jax.experimental import pallas as pl
from jax.experimental.pallas import tpu as pltpu
from jax.experimental.pallas import tpu_sc as plsc

BATCH = 1024
SEQ = 200
EMBED_DIM = 64
SCALE = 8.0

NUM_CORES = 2
NUM_SUBCORES = 16
NW = NUM_CORES * NUM_SUBCORES
ROWS_PER_W = BATCH * SEQ // NW
CHUNK = 128
NCHUNK = ROWS_PER_W // CHUNK
NBUF = 5
PE_ROWS = SEQ + CHUNK
VREGS_PER_ROW = EMBED_DIM // 16


def _pe_kernel_body(src_hbm, pe_hbm, table_hbm, out_hbm, idx_v, pe_v, *bufs):
    ins = bufs[0:NBUF]
    outs = bufs[NBUF : 2 * NBUF]
    sin = bufs[2 * NBUF : 3 * NBUF]
    sout = bufs[3 * NBUF : 4 * NBUF]

    wid = lax.axis_index("s") * NUM_CORES + lax.axis_index("c")
    base = wid * ROWS_PER_W

    pltpu.sync_copy(src_hbm.at[wid], idx_v)
    pltpu.sync_copy(pe_hbm, pe_v)

    for b in range(NBUF):
        pltpu.async_copy(table_hbm.at[idx_v.at[b]], ins[b], sin[b])

    def block_body(g0, _):
        g = g0 * NBUF
        for b in range(NBUF):
            c = g + b
            pltpu.make_async_copy(table_hbm.at[idx_v.at[c]], ins[b], sin[b]).wait()
            @pl.when(c >= NBUF)
            def _():
                pltpu.make_async_copy(
                    out_hbm.at[pl.ds(0, CHUNK)], outs[b], sout[b]
                ).wait()

            pstart = lax.rem(c * CHUNK, SEQ)

            def row_body(r, _):
                p = pstart + r
                for d in range(VREGS_PER_ROW):
                    o = d * 16
                    outs[b][r, pl.ds(o, 16)] = (
                        ins[b][r, pl.ds(o, 16)] * SCALE + pe_v[p, pl.ds(o, 16)]
                    )
                return ()

            lax.fori_loop(0, CHUNK, row_body, (), unroll=4)

            pltpu.async_copy(outs[b], out_hbm.at[pl.ds(base + c * CHUNK, CHUNK)], sout[b])

            @pl.when(c + NBUF < NCHUNK)
            def _():
                pltpu.async_copy(table_hbm.at[idx_v.at[c + NBUF]], ins[b], sin[b])

        return ()

    lax.fori_loop(0, NCHUNK // NBUF, block_body, ())

    for b in range(NBUF):
        pltpu.make_async_copy(out_hbm.at[pl.ds(0, CHUNK)], outs[b], sout[b]).wait()


@jax.jit
def kernel(src, table, pe):
    src_r = src.reshape(NW, NCHUNK, CHUNK)
    pe_seq = pe[:SEQ]
    pe_ext = jnp.concatenate([pe_seq, pe_seq[:CHUNK]], axis=0)

    mesh = plsc.VectorSubcoreMesh(core_axis_name="c", subcore_axis_name="s")
    out = pl.kernel(
        _pe_kernel_body,
        out_type=jax.ShapeDtypeStruct((BATCH * SEQ, EMBED_DIM), jnp.float32),
        mesh=mesh,
        compiler_params=pltpu.CompilerParams(
            use_tc_tiling_on_sc=False, skip_device_barrier=True
        ),
        scratch_types=(
            [
                pltpu.VMEM((NCHUNK, CHUNK), jnp.int32),
                pltpu.VMEM((PE_ROWS, EMBED_DIM), jnp.float32),
            ]
            + [pltpu.VMEM((CHUNK, EMBED_DIM), jnp.float32) for _ in range(2 * NBUF)]
            + [pltpu.SemaphoreType.DMA for _ in range(2 * NBUF)]
        ),
    )(src_r, pe_ext, table)
    return out.reshape(BATCH, SEQ, EMBED_DIM)


# speedup vs baseline: 1.4001x; 1.4001x over previous
import jax
import jax.numpy as jnp
from jax import lax
from jax.experimental import pallas as pl
from jax.experimental.pallas import tpu as pltpu
from jax.experimental.pallas import tpu_sc as plsc
from jax.experimental.layout import Layout, Format, with_layout_constraint

BATCH = 1024
SEQ = 200
EMBED_DIM = 64
SCALE = 8.0

NUM_CORES = 2
NUM_SUBCORES = 16
NW = NUM_CORES * NUM_SUBCORES
ROWS_PER_W = BATCH * SEQ // NW
CHUNK = 128
NCHUNK = ROWS_PER_W // CHUNK
NBUF = 5
PE_ROWS = SEQ + CHUNK
VREGS_PER_ROW = EMBED_DIM // 16


def _pe_kernel_body(src_hbm, pe_hbm, table_hbm, out_hbm, idx_v, pe_v, *bufs):
    ins = bufs[0:NBUF]
    outs = bufs[NBUF : 2 * NBUF]
    sin = bufs[2 * NBUF : 3 * NBUF]
    sout = bufs[3 * NBUF : 4 * NBUF]

    wid = lax.axis_index("s") * NUM_CORES + lax.axis_index("c")
    base = wid * ROWS_PER_W

    pltpu.sync_copy(src_hbm.at[wid], idx_v)
    pltpu.sync_copy(pe_hbm, pe_v)

    for b in range(NBUF):
        pltpu.async_copy(table_hbm.at[idx_v.at[b]], ins[b], sin[b])

    def block_body(g0, _):
        g = g0 * NBUF
        for b in range(NBUF):
            c = g + b
            pltpu.make_async_copy(table_hbm.at[idx_v.at[c]], ins[b], sin[b]).wait()
            @pl.when(c >= NBUF)
            def _():
                pltpu.make_async_copy(
                    out_hbm.at[pl.ds(0, CHUNK)], outs[b], sout[b]
                ).wait()

            pstart = lax.rem(c * CHUNK, SEQ)

            def row_body(r, _):
                p = pstart + r
                for d in range(VREGS_PER_ROW):
                    o = d * 16
                    outs[b][r, pl.ds(o, 16)] = (
                        ins[b][r, pl.ds(o, 16)] * SCALE + pe_v[p, pl.ds(o, 16)]
                    )
                return ()

            lax.fori_loop(0, CHUNK, row_body, (), unroll=4)

            pltpu.async_copy(outs[b], out_hbm.at[pl.ds(base + c * CHUNK, CHUNK)], sout[b])

            @pl.when(c + NBUF < NCHUNK)
            def _():
                pltpu.async_copy(table_hbm.at[idx_v.at[c + NBUF]], ins[b], sin[b])

        return ()

    lax.fori_loop(0, NCHUNK // NBUF, block_body, ())

    for b in range(NBUF):
        pltpu.make_async_copy(out_hbm.at[pl.ds(0, CHUNK)], outs[b], sout[b]).wait()


def _sc_rowmajor(shape):
    return Layout(major_to_minor=tuple(range(len(shape))), tiling=((8,),))


@jax.jit
def kernel(src, table, pe):
    src_r = src.reshape(NW, NCHUNK, CHUNK)
    pe_seq = pe[:SEQ]
    pe_ext = jnp.concatenate([pe_seq, pe_seq[:CHUNK]], axis=0)
    tableL = with_layout_constraint(table, _sc_rowmajor(table.shape))

    mesh = plsc.VectorSubcoreMesh(core_axis_name="c", subcore_axis_name="s")
    out = pl.kernel(
        _pe_kernel_body,
        out_type=jax.ShapeDtypeStruct((BATCH * SEQ, EMBED_DIM), jnp.float32),
        mesh=mesh,
        compiler_params=pltpu.CompilerParams(use_tc_tiling_on_sc=False),
        scratch_types=(
            [
                pltpu.VMEM((NCHUNK, CHUNK), jnp.int32),
                pltpu.VMEM((PE_ROWS, EMBED_DIM), jnp.float32),
            ]
            + [pltpu.VMEM((CHUNK, EMBED_DIM), jnp.float32) for _ in range(2 * NBUF)]
            + [pltpu.SemaphoreType.DMA for _ in range(2 * NBUF)]
        ),
    )(src_r, pe_ext, tableL)
    return out.reshape(BATCH, SEQ, EMBED_DIM)
